# baseline (speedup 1.0000x reference)
import jax
import jax.numpy as jnp
from jax import lax
from jax.experimental import pallas as pl
from jax.experimental.pallas import tpu as pltpu
from jax.experimental.pallas import tpu_sc as plsc

NC = 2
NS = 16


def _npad(N):
  return -(-N // (NS * 128)) * (NS * 128)


def _make_sc_scatter(N, E, W, has_gather):
  NW = NC * NS
  EW = E // NW
  CH = EW // 128
  TAIL = EW - CH * 128
  CHUNK = 16 if has_gather else 128
  NBUF = 12
  HPR = 128 // CHUNK
  IF = 8
  NPAD = _npad(N)
  assert E % NW == 0 and TAIL % 16 == 0 and CH >= IF and TAIL <= CHUNK
  assert not has_gather or (HPR * CH) % NBUF == 0
  RPS = NPAD // NS
  NPIECE = RPS // CHUNK
  mesh = plsc.VectorSubcoreMesh(core_axis_name="c", subcore_axis_name="s")

  scratch = [
      pltpu.VMEM((CH, 128), jnp.int32),
      pltpu.VMEM((TAIL,), jnp.int32),
      pltpu.VMEM((CHUNK, W), jnp.float32),
      pltpu.VMEM_SHARED((NPAD, W), jnp.float32),
      pltpu.SemaphoreType.DMA,
      pltpu.SemaphoreType.DMA,
  ]
  if has_gather:
    scratch = [
        pltpu.VMEM((CH, 128), jnp.int32),
        pltpu.VMEM((TAIL,), jnp.int32),
    ] + [pltpu.VMEM((CHUNK, W), jnp.float32)] * (NBUF - 1) + [
        pltpu.SemaphoreType.DMA] * NBUF + [
        pltpu.SemaphoreType.DMA] * (NBUF - 1
    ) + scratch

  def body(*refs):
    if has_gather:
      (gidxm_hbm, gidxt_hbm, sidxm_hbm, sidxt_hbm, table_hbm, out_hbm,
       gidx_all, gidxT) = refs[:8]
      rows_hi = refs[8:8 + NBUF - 1]
      sg = list(refs[8 + NBUF - 1:8 + 2 * NBUF - 1])
      ss_hi = refs[8 + 2 * NBUF - 1:8 + 3 * NBUF - 2]
      (sidx_all, sidxT, rows0, acc, ss0, ws) = refs[8 + 3 * NBUF - 2:]
      rows = [rows0] + list(rows_hi)
      ss = [ss0] + list(ss_hi)
    else:
      (sidxm_hbm, sidxt_hbm, ones_hbm, out_hbm,
       sidx_all, sidxT, rows0, acc, ss0, ws) = refs
    c = lax.axis_index("c")
    s = lax.axis_index("s")
    wid = c * NS + s
    lo = s * RPS

    zero = jnp.zeros((16,), jnp.float32)
    def zrow(r, carry):
      for j in range(W // 16):
        rows0[r, pl.ds(j * 16, 16)] = zero
      return carry
    lax.fori_loop(0, CHUNK, zrow, 0)
    def zpiece(j, carry):
      pltpu.async_copy(rows0, acc.at[pl.ds(lo + j * CHUNK, CHUNK)], ws)
      @pl.when(j > 0)
      def _():
        pltpu.make_async_copy(
            rows0, acc.at[pl.ds(lo + (j - 1) * CHUNK, CHUNK)], ws).wait()
      return carry
    lax.fori_loop(0, NPIECE, zpiece, 0)
    pltpu.async_copy(sidxm_hbm.at[wid], sidx_all, ss0)
    if has_gather:
      pltpu.async_copy(sidxt_hbm.at[wid], sidxT, ss[1])
      pltpu.async_copy(gidxm_hbm.at[wid], gidx_all, sg[0])
      pltpu.async_copy(gidxt_hbm.at[wid], gidxT, sg[1])
      pltpu.make_async_copy(gidxm_hbm.at[wid], gidx_all, sg[0]).wait()
      pltpu.make_async_copy(gidxt_hbm.at[wid], gidxT, sg[1]).wait()
      pltpu.make_async_copy(sidxt_hbm.at[wid], sidxT, ss[1]).wait()
    else:
      pltpu.async_copy(sidxt_hbm.at[wid], sidxT, ss0)
      pltpu.make_async_copy(sidxt_hbm.at[wid], sidxT, ss0).wait()
    pltpu.make_async_copy(sidxm_hbm.at[wid], sidx_all, ss0).wait()
    pltpu.make_async_copy(
        rows0, acc.at[pl.ds(lo + (NPIECE - 1) * CHUNK, CHUNK)], ws).wait()
    if not has_gather:
      pltpu.sync_copy(ones_hbm, rows0)
    if has_gather:
      for k in range(NBUF):
        pltpu.async_copy(
            table_hbm.at[gidx_all.at[k // HPR,
                                     pl.ds(CHUNK * (k % HPR), CHUNK)]],
            rows[k], sg[k])
    plsc.subcore_barrier()

    if has_gather:
      T = (HPR * CH) // NBUF
      def chunk_at(idx, q):
        return idx.at[lax.div(q, HPR), pl.ds(lax.rem(q, HPR) * CHUNK, CHUNK)]
      def ebody(t, carry):
        q0 = t * NBUF
        for k in range(NBUF):
          pltpu.make_async_copy(
              table_hbm.at[chunk_at(gidx_all, q0 + k)], rows[k],
              sg[k]).wait()
          pltpu.async_copy(rows[k], acc.at[chunk_at(sidx_all, q0 + k)],
                           ss[k], add=True)
        for k in range(NBUF):
          pltpu.make_async_copy(
              rows[k], acc.at[chunk_at(sidx_all, q0 + k)], ss[k]).wait()
          @pl.when(t + 1 < T)
          def _():
            pltpu.async_copy(
                table_hbm.at[chunk_at(gidx_all, q0 + NBUF + k)], rows[k],
                sg[k])
        return carry
      lax.fori_loop(0, T, ebody, 0)
      pltpu.sync_copy(table_hbm.at[gidxT], rows0.at[pl.ds(0, TAIL)])
      pltpu.sync_copy(rows0.at[pl.ds(0, TAIL)], acc.at[sidxT], add=True)
    else:
      for j in range(IF):
        pltpu.async_copy(rows0, acc.at[sidx_all.at[j]], ss0, add=True)
      def ebody(i, carry):
        pltpu.make_async_copy(rows0, acc.at[sidx_all.at[i - IF]], ss0).wait()
        pltpu.async_copy(rows0, acc.at[sidx_all.at[i]], ss0, add=True)
        return carry
      lax.fori_loop(IF, CH, ebody, 0)
      for j in range(IF):
        pltpu.make_async_copy(rows0, acc.at[sidx_all.at[CH - IF + j]],
                              ss0).wait()
      pltpu.sync_copy(rows0.at[pl.ds(0, TAIL)], acc.at[sidxT], add=True)
    plsc.subcore_barrier()

    def wpiece(j, carry):
      pltpu.async_copy(acc.at[pl.ds(lo + j * CHUNK, CHUNK)],
                       out_hbm.at[c, pl.ds(lo + j * CHUNK, CHUNK)], ws)
      @pl.when(j > 0)
      def _():
        pltpu.make_async_copy(acc.at[pl.ds(lo + (j - 1) * CHUNK, CHUNK)],
                              out_hbm.at[c, pl.ds(lo + (j - 1) * CHUNK,
                                                  CHUNK)], ws).wait()
      return carry
    lax.fori_loop(0, NPIECE, wpiece, 0)
    pltpu.make_async_copy(acc.at[pl.ds(lo + (NPIECE - 1) * CHUNK, CHUNK)],
                          out_hbm.at[c, pl.ds(lo + (NPIECE - 1) * CHUNK,
                                              CHUNK)], ws).wait()

  return pl.kernel(
      body,
      out_type=jax.ShapeDtypeStruct((NC, NPAD, W), jnp.float32),
      mesh=mesh,
      scratch_types=scratch,
  )


_R = 1000


def _tc1(x, W1, degp):
  N, D = x.shape

  def body(x_ref, w_ref, degp_ref, h_ref, hs_ref, d128_ref):
    h = jnp.dot(x_ref[...], w_ref[...], preferred_element_type=jnp.float32)
    deg = degp_ref[0, :, :1] + degp_ref[1, :, :1] + 1.0
    dcol = lax.rsqrt(deg)
    h_ref[...] = h
    hs_ref[...] = h * dcol
    d128_ref[...] = jnp.broadcast_to(dcol, (_R, D))

  G = N // _R
  return pl.pallas_call(
      body,
      grid=(G,),
      in_specs=[
          pl.BlockSpec((_R, D), lambda i: (i, 0)),
          pl.BlockSpec((D, D), lambda i: (0, 0)),
          pl.BlockSpec((NC, _R, D), lambda i: (0, i, 0)),
      ],
      out_specs=[
          pl.BlockSpec((_R, D), lambda i: (i, 0)),
          pl.BlockSpec((_R, D), lambda i: (i, 0)),
          pl.BlockSpec((_R, D), lambda i: (i, 0)),
      ],
      out_shape=[
          jax.ShapeDtypeStruct((N, D), jnp.float32),
          jax.ShapeDtypeStruct((N, D), jnp.float32),
          jax.ShapeDtypeStruct((N, D), jnp.float32),
      ],
  )(x, W1, degp)


def _tc2(aggp, h_prev, d128, b, W):
  N, D = h_prev.shape

  def body(aggp_ref, h_ref, d_ref, b_ref, w_ref, h2_ref, hs2_ref):
    d = d_ref[...]
    agg = aggp_ref[0] + aggp_ref[1]
    hp = jnp.maximum(d * agg + d * d * h_ref[...] + b_ref[...], 0.0)
    h2 = jnp.dot(hp, w_ref[...], preferred_element_type=jnp.float32)
    h2_ref[...] = h2
    hs2_ref[...] = d * h2

  G = N // _R
  return pl.pallas_call(
      body,
      grid=(G,),
      in_specs=[
          pl.BlockSpec((NC, _R, D), lambda i: (0, i, 0)),
          pl.BlockSpec((_R, D), lambda i: (i, 0)),
          pl.BlockSpec((_R, D), lambda i: (i, 0)),
          pl.BlockSpec((1, D), lambda i: (0, 0)),
          pl.BlockSpec((D, D), lambda i: (0, 0)),
      ],
      out_specs=[
          pl.BlockSpec((_R, D), lambda i: (i, 0)),
          pl.BlockSpec((_R, D), lambda i: (i, 0)),
      ],
      out_shape=[
          jax.ShapeDtypeStruct((N, D), jnp.float32),
          jax.ShapeDtypeStruct((N, D), jnp.float32),
      ],
  )(aggp, h_prev, d128, b, W)


def _tc4(aggp, h3, d128, b3p):
  N, D = h3.shape
  G = N // _R

  def body(aggp_ref, h_ref, d_ref, b_ref, out_ref, vacc):
    i = pl.program_id(0)
    d = d_ref[...]
    agg = aggp_ref[0] + aggp_ref[1]
    t = d * agg + d * d * h_ref[...]
    part = jnp.sum(t, axis=0, keepdims=True)

    @pl.when(i == 0)
    def _():
      vacc[...] = jnp.zeros_like(vacc)

    vacc[...] += part

    @pl.when(i == G - 1)
    def _():
      out_ref[...] = vacc[...] * (1.0 / N) + b_ref[...]

  return pl.pallas_call(
      body,
      grid=(G,),
      in_specs=[
          pl.BlockSpec((NC, _R, D), lambda i: (0, i, 0)),
          pl.BlockSpec((_R, D), lambda i: (i, 0)),
          pl.BlockSpec((_R, D), lambda i: (i, 0)),
          pl.BlockSpec((1, D), lambda i: (0, 0)),
      ],
      out_specs=pl.BlockSpec((1, D), lambda i: (0, 0)),
      out_shape=jax.ShapeDtypeStruct((1, D), jnp.float32),
      scratch_shapes=[pltpu.VMEM((1, D), jnp.float32)],
  )(aggp, h3, d128, b3p)


def kernel(x, edge_index, W1, b1, W2, b2, W3, b3):
  N, D = x.shape
  E = edge_index.shape[1]
  DO = W3.shape[1]
  src = edge_index[0].astype(jnp.int32)
  dst = edge_index[1].astype(jnp.int32)

  deg_sc = _make_sc_scatter(N, E, D, has_gather=False)
  agg_sc = _make_sc_scatter(N, E, D, has_gather=True)

  W3p = jnp.pad(W3, ((0, 0), (0, D - DO)))
  b3p = jnp.pad(b3, (0, D - DO)).reshape(1, D)

  NW = NC * NS
  EW = E // NW
  CH = EW // 128
  srcr = src.reshape(NW, EW)
  dstr = dst.reshape(NW, EW)
  src_m = srcr[:, :CH * 128].reshape(NW, CH, 128)
  src_t = srcr[:, CH * 128:]
  dst_m = dstr[:, :CH * 128].reshape(NW, CH, 128)
  dst_t = dstr[:, CH * 128:]
  ones = jnp.ones((128, D), jnp.float32)

  degp = deg_sc(dst_m, dst_t, ones)
  h1, hs1, d128 = _tc1(x, W1, degp)
  aggp1 = agg_sc(src_m, src_t, dst_m, dst_t, hs1)
  h2, hs2 = _tc2(aggp1, h1, d128, b1.reshape(1, -1), W2)
  aggp2 = agg_sc(src_m, src_t, dst_m, dst_t, hs2)
  h3, hs3 = _tc2(aggp2, h2, d128, b2.reshape(1, -1), W3p)
  aggp3 = agg_sc(src_m, src_t, dst_m, dst_t, hs3)
  outp = _tc4(aggp3, h3, d128, b3p)
  return outp[:, :DO]


# speedup vs baseline: 1.1050x; 1.1050x over previous
import jax
import jax.numpy as jnp
from jax import lax
from jax.experimental import pallas as pl
from jax.experimental.pallas import tpu as pltpu
from jax.experimental.pallas import tpu_sc as plsc

NC = 2
NS = 16


def _npad(N):
  return -(-N // (NS * 128)) * (NS * 128)


def _make_sc_scatter(N, E, W, has_gather):
  NW = NC * NS
  EW = E // NW
  CH = EW // 128
  TAIL = EW - CH * 128
  CHUNK = 32 if has_gather else 128
  NBUF = 6
  HPR = 128 // CHUNK
  IF = 8
  NPAD = _npad(N)
  assert E % NW == 0 and TAIL % 16 == 0 and CH >= IF and TAIL <= CHUNK
  assert not has_gather or (HPR * CH) % NBUF == 0
  RPS = NPAD // NS
  NPIECE = RPS // CHUNK
  mesh = plsc.VectorSubcoreMesh(core_axis_name="c", subcore_axis_name="s")

  scratch = [
      pltpu.VMEM((CH, 128), jnp.int32),
      pltpu.VMEM((TAIL,), jnp.int32),
      pltpu.VMEM((CHUNK, W), jnp.float32),
      pltpu.VMEM_SHARED((NPAD, W), jnp.float32),
      pltpu.SemaphoreType.DMA,
      pltpu.SemaphoreType.DMA,
  ]
  if has_gather:
    scratch = [
        pltpu.VMEM((CH, 128), jnp.int32),
        pltpu.VMEM((TAIL,), jnp.int32),
    ] + [pltpu.VMEM((CHUNK, W), jnp.float32)] * (NBUF - 1) + [
        pltpu.SemaphoreType.DMA] * NBUF + [
        pltpu.SemaphoreType.DMA] * (NBUF - 1
    ) + scratch

  def body(*refs):
    if has_gather:
      (gidxm_hbm, gidxt_hbm, sidxm_hbm, sidxt_hbm, table_hbm, out_hbm,
       gidx_all, gidxT) = refs[:8]
      rows_hi = refs[8:8 + NBUF - 1]
      sg = list(refs[8 + NBUF - 1:8 + 2 * NBUF - 1])
      ss_hi = refs[8 + 2 * NBUF - 1:8 + 3 * NBUF - 2]
      (sidx_all, sidxT, rows0, acc, ss0, ws) = refs[8 + 3 * NBUF - 2:]
      rows = [rows0] + list(rows_hi)
      ss = [ss0] + list(ss_hi)
    else:
      (sidxm_hbm, sidxt_hbm, ones_hbm, out_hbm,
       sidx_all, sidxT, rows0, acc, ss0, ws) = refs
    c = lax.axis_index("c")
    s = lax.axis_index("s")
    wid = c * NS + s
    lo = s * RPS

    zero = jnp.zeros((16,), jnp.float32)
    def zrow(r, carry):
      for j in range(W // 16):
        rows0[r, pl.ds(j * 16, 16)] = zero
      return carry
    lax.fori_loop(0, CHUNK, zrow, 0)
    def zpiece(j, carry):
      pltpu.async_copy(rows0, acc.at[pl.ds(lo + j * CHUNK, CHUNK)], ws)
      @pl.when(j > 0)
      def _():
        pltpu.make_async_copy(
            rows0, acc.at[pl.ds(lo + (j - 1) * CHUNK, CHUNK)], ws).wait()
      return carry
    lax.fori_loop(0, NPIECE, zpiece, 0)
    pltpu.async_copy(sidxm_hbm.at[wid], sidx_all, ss0)
    if has_gather:
      pltpu.async_copy(sidxt_hbm.at[wid], sidxT, ss[1])
      pltpu.async_copy(gidxm_hbm.at[wid], gidx_all, sg[0])
      pltpu.async_copy(gidxt_hbm.at[wid], gidxT, sg[1])
      pltpu.make_async_copy(gidxm_hbm.at[wid], gidx_all, sg[0]).wait()
      pltpu.make_async_copy(gidxt_hbm.at[wid], gidxT, sg[1]).wait()
      pltpu.make_async_copy(sidxt_hbm.at[wid], sidxT, ss[1]).wait()
    else:
      pltpu.async_copy(sidxt_hbm.at[wid], sidxT, ss0)
      pltpu.make_async_copy(sidxt_hbm.at[wid], sidxT, ss0).wait()
    pltpu.make_async_copy(sidxm_hbm.at[wid], sidx_all, ss0).wait()
    pltpu.make_async_copy(
        rows0, acc.at[pl.ds(lo + (NPIECE - 1) * CHUNK, CHUNK)], ws).wait()
    if not has_gather:
      pltpu.sync_copy(ones_hbm, rows0)
    if has_gather:
      for k in range(NBUF):
        pltpu.async_copy(
            table_hbm.at[gidx_all.at[k // HPR,
                                     pl.ds(CHUNK * (k % HPR), CHUNK)]],
            rows[k], sg[k])
    plsc.subcore_barrier()

    if has_gather:
      T = (HPR * CH) // NBUF
      def chunk_at(idx, q):
        return idx.at[lax.div(q, HPR), pl.ds(lax.rem(q, HPR) * CHUNK, CHUNK)]
      def ebody(t, carry):
        q0 = t * NBUF
        for k in range(NBUF):
          pltpu.make_async_copy(
              table_hbm.at[chunk_at(gidx_all, q0 + k)], rows[k],
              sg[k]).wait()
          pltpu.async_copy(rows[k], acc.at[chunk_at(sidx_all, q0 + k)],
                           ss[k], add=True)
        for k in range(NBUF):
          pltpu.make_async_copy(
              rows[k], acc.at[chunk_at(sidx_all, q0 + k)], ss[k]).wait()
          @pl.when(t + 1 < T)
          def _():
            pltpu.async_copy(
                table_hbm.at[chunk_at(gidx_all, q0 + NBUF + k)], rows[k],
                sg[k])
        return carry
      lax.fori_loop(0, T, ebody, 0)
      pltpu.sync_copy(table_hbm.at[gidxT], rows0.at[pl.ds(0, TAIL)])
      pltpu.sync_copy(rows0.at[pl.ds(0, TAIL)], acc.at[sidxT], add=True)
    else:
      for j in range(IF):
        pltpu.async_copy(rows0, acc.at[sidx_all.at[j]], ss0, add=True)
      def ebody(i, carry):
        pltpu.make_async_copy(rows0, acc.at[sidx_all.at[i - IF]], ss0).wait()
        pltpu.async_copy(rows0, acc.at[sidx_all.at[i]], ss0, add=True)
        return carry
      lax.fori_loop(IF, CH, ebody, 0)
      for j in range(IF):
        pltpu.make_async_copy(rows0, acc.at[sidx_all.at[CH - IF + j]],
                              ss0).wait()
      pltpu.sync_copy(rows0.at[pl.ds(0, TAIL)], acc.at[sidxT], add=True)
    plsc.subcore_barrier()

    def wpiece(j, carry):
      pltpu.async_copy(acc.at[pl.ds(lo + j * CHUNK, CHUNK)],
                       out_hbm.at[c, pl.ds(lo + j * CHUNK, CHUNK)], ws)
      @pl.when(j > 0)
      def _():
        pltpu.make_async_copy(acc.at[pl.ds(lo + (j - 1) * CHUNK, CHUNK)],
                              out_hbm.at[c, pl.ds(lo + (j - 1) * CHUNK,
                                                  CHUNK)], ws).wait()
      return carry
    lax.fori_loop(0, NPIECE, wpiece, 0)
    pltpu.make_async_copy(acc.at[pl.ds(lo + (NPIECE - 1) * CHUNK, CHUNK)],
                          out_hbm.at[c, pl.ds(lo + (NPIECE - 1) * CHUNK,
                                              CHUNK)], ws).wait()

  return pl.kernel(
      body,
      out_type=jax.ShapeDtypeStruct((NC, NPAD, W), jnp.float32),
      mesh=mesh,
      scratch_types=scratch,
  )


_R = 1000


def _tc1(x, W1, degp):
  N, D = x.shape

  def body(x_ref, w_ref, degp_ref, h_ref, hs_ref, d128_ref):
    h = jnp.dot(x_ref[...], w_ref[...], preferred_element_type=jnp.float32)
    deg = degp_ref[0, :, :1] + degp_ref[1, :, :1] + 1.0
    dcol = lax.rsqrt(deg)
    h_ref[...] = h
    hs_ref[...] = h * dcol
    d128_ref[...] = jnp.broadcast_to(dcol, (_R, D))

  G = N // _R
  return pl.pallas_call(
      body,
      grid=(G,),
      in_specs=[
          pl.BlockSpec((_R, D), lambda i: (i, 0)),
          pl.BlockSpec((D, D), lambda i: (0, 0)),
          pl.BlockSpec((NC, _R, D), lambda i: (0, i, 0)),
      ],
      out_specs=[
          pl.BlockSpec((_R, D), lambda i: (i, 0)),
          pl.BlockSpec((_R, D), lambda i: (i, 0)),
          pl.BlockSpec((_R, D), lambda i: (i, 0)),
      ],
      out_shape=[
          jax.ShapeDtypeStruct((N, D), jnp.float32),
          jax.ShapeDtypeStruct((N, D), jnp.float32),
          jax.ShapeDtypeStruct((N, D), jnp.float32),
      ],
  )(x, W1, degp)


def _tc2(aggp, h_prev, d128, b, W):
  N, D = h_prev.shape

  def body(aggp_ref, h_ref, d_ref, b_ref, w_ref, h2_ref, hs2_ref):
    d = d_ref[...]
    agg = aggp_ref[0] + aggp_ref[1]
    hp = jnp.maximum(d * agg + d * d * h_ref[...] + b_ref[...], 0.0)
    h2 = jnp.dot(hp, w_ref[...], preferred_element_type=jnp.float32)
    h2_ref[...] = h2
    hs2_ref[...] = d * h2

  G = N // _R
  return pl.pallas_call(
      body,
      grid=(G,),
      in_specs=[
          pl.BlockSpec((NC, _R, D), lambda i: (0, i, 0)),
          pl.BlockSpec((_R, D), lambda i: (i, 0)),
          pl.BlockSpec((_R, D), lambda i: (i, 0)),
          pl.BlockSpec((1, D), lambda i: (0, 0)),
          pl.BlockSpec((D, D), lambda i: (0, 0)),
      ],
      out_specs=[
          pl.BlockSpec((_R, D), lambda i: (i, 0)),
          pl.BlockSpec((_R, D), lambda i: (i, 0)),
      ],
      out_shape=[
          jax.ShapeDtypeStruct((N, D), jnp.float32),
          jax.ShapeDtypeStruct((N, D), jnp.float32),
      ],
  )(aggp, h_prev, d128, b, W)


def _tc4(aggp, h3, d128, b3p):
  N, D = h3.shape
  G = N // _R

  def body(aggp_ref, h_ref, d_ref, b_ref, out_ref, vacc):
    i = pl.program_id(0)
    d = d_ref[...]
    agg = aggp_ref[0] + aggp_ref[1]
    t = d * agg + d * d * h_ref[...]
    part = jnp.sum(t, axis=0, keepdims=True)

    @pl.when(i == 0)
    def _():
      vacc[...] = jnp.zeros_like(vacc)

    vacc[...] += part

    @pl.when(i == G - 1)
    def _():
      out_ref[...] = vacc[...] * (1.0 / N) + b_ref[...]

  return pl.pallas_call(
      body,
      grid=(G,),
      in_specs=[
          pl.BlockSpec((NC, _R, D), lambda i: (0, i, 0)),
          pl.BlockSpec((_R, D), lambda i: (i, 0)),
          pl.BlockSpec((_R, D), lambda i: (i, 0)),
          pl.BlockSpec((1, D), lambda i: (0, 0)),
      ],
      out_specs=pl.BlockSpec((1, D), lambda i: (0, 0)),
      out_shape=jax.ShapeDtypeStruct((1, D), jnp.float32),
      scratch_shapes=[pltpu.VMEM((1, D), jnp.float32)],
  )(aggp, h3, d128, b3p)


def kernel(x, edge_index, W1, b1, W2, b2, W3, b3):
  N, D = x.shape
  E = edge_index.shape[1]
  DO = W3.shape[1]
  src = edge_index[0].astype(jnp.int32)
  dst = edge_index[1].astype(jnp.int32)

  deg_sc = _make_sc_scatter(N, E, D, has_gather=False)
  agg_sc = _make_sc_scatter(N, E, D, has_gather=True)

  W3p = jnp.pad(W3, ((0, 0), (0, D - DO)))
  b3p = jnp.pad(b3, (0, D - DO)).reshape(1, D)

  NW = NC * NS
  EW = E // NW
  CH = EW // 128
  srcr = src.reshape(NW, EW)
  dstr = dst.reshape(NW, EW)
  src_m = srcr[:, :CH * 128].reshape(NW, CH, 128)
  src_t = srcr[:, CH * 128:]
  dst_m = dstr[:, :CH * 128].reshape(NW, CH, 128)
  dst_t = dstr[:, CH * 128:]
  ones = jnp.ones((128, D), jnp.float32)

  degp = deg_sc(dst_m, dst_t, ones)
  h1, hs1, d128 = _tc1(x, W1, degp)
  aggp1 = agg_sc(src_m, src_t, dst_m, dst_t, hs1)
  h2, hs2 = _tc2(aggp1, h1, d128, b1.reshape(1, -1), W2)
  aggp2 = agg_sc(src_m, src_t, dst_m, dst_t, hs2)
  h3, hs3 = _tc2(aggp2, h2, d128, b2.reshape(1, -1), W3p)
  aggp3 = agg_sc(src_m, src_t, dst_m, dst_t, hs3)
  outp = _tc4(aggp3, h3, d128, b3p)
  return outp[:, :DO]
